# Initial kernel scaffold; baseline (speedup 1.0000x reference)
import jax
import jax.numpy as jnp
from jax.experimental import pallas as pl


def kernel(x, embedding_weight):
    raise NotImplementedError("write your pallas kernel here")


# speedup vs baseline: 3.4732x; 3.4732x over previous
import functools

import jax
import jax.numpy as jnp
from jax import lax
from jax.experimental import pallas as pl
from jax.experimental.pallas import tpu as pltpu
from jax.experimental.pallas import tpu_sc as plsc

N_TOK = 9216
N_CODE = 8192
DIM = 64
COMMIT = 0.25

TM = 256
TN = 512
NT = N_TOK // TM
NCB = N_CODE // TN

NW = 32
BPW = N_TOK // NW
CHUNK = 72
NCHUNK = BPW // CHUNK


def _argmin_body(x_ref, e_ref, idx_ref, minv_ref, mini_ref):
    cb = pl.program_id(1)

    @pl.when(cb == 0)
    def _init():
        minv_ref[...] = jnp.full((TM, 1), jnp.inf, jnp.float32)
        mini_ref[...] = jnp.zeros((TM, 1), jnp.int32)

    xb = x_ref[...]
    eb = e_ref[...]
    xn = jnp.sum(xb * xb, axis=1, keepdims=True)
    en = jnp.sum(eb * eb, axis=1)
    m = lax.dot_general(xb, eb, (((1,), (1,)), ((), ())),
                        precision=lax.Precision.DEFAULT,
                        preferred_element_type=jnp.float32)
    d = (xn + en[None, :]) - 2.0 * m

    bmin = jnp.min(d, axis=1, keepdims=True)
    ii = lax.broadcasted_iota(jnp.int32, (TM, TN), 1)
    big = jnp.int32(2**31 - 1)
    bidx = jnp.min(jnp.where(d == bmin, ii, big), axis=1, keepdims=True)
    bidx = bidx + cb * TN

    better = bmin < minv_ref[...]
    minv_ref[...] = jnp.where(better, bmin, minv_ref[...])
    mini_ref[...] = jnp.where(better, bidx, mini_ref[...])

    @pl.when(cb == NCB - 1)
    def _flush():
        idx_ref[...] = mini_ref[...]


def _argmin_call(x, e, interpret=False):
    return pl.pallas_call(
        _argmin_body,
        grid=(NT, NCB),
        in_specs=[
            pl.BlockSpec((TM, DIM), lambda t, c: (t, 0)),
            pl.BlockSpec((TN, DIM), lambda t, c: (c, 0)),
        ],
        out_specs=pl.BlockSpec((TM, 1), lambda t, c: (t, 0)),
        out_shape=jax.ShapeDtypeStruct((N_TOK, 1), jnp.int32),
        scratch_shapes=[
            pltpu.VMEM((TM, 1), jnp.float32),
            pltpu.VMEM((TM, 1), jnp.int32),
        ],
        interpret=interpret,
    )(x, e)


def _sc_body(table_hbm, idx_hbm, ones_hbm, zeros_hbm, q_hbm, counts_hbm,
             idx_v, rows_v, ones_v, hist_sh, sem):
    cid = lax.axis_index("c")
    sid = lax.axis_index("s")
    wid = sid * 2 + cid

    pltpu.sync_copy(idx_hbm.at[pl.ds(wid * NCHUNK, NCHUNK)], idx_v)
    for j in range(NCHUNK):
        pltpu.async_copy(table_hbm.at[idx_v.at[j]],
                         rows_v.at[pl.ds(j * CHUNK, CHUNK)], sem).wait()
    pltpu.sync_copy(rows_v, q_hbm.at[pl.ds(wid * BPW, BPW)])

    rows_per_sub = N_CODE // 16
    pltpu.sync_copy(zeros_hbm,
                    hist_sh.at[pl.ds(sid * rows_per_sub, rows_per_sub)])
    pltpu.sync_copy(ones_hbm, ones_v)
    plsc.subcore_barrier()
    for j in range(NCHUNK):
        pltpu.sync_copy(ones_v, hist_sh.at[idx_v.at[j]], add=True)
    plsc.subcore_barrier()
    pltpu.sync_copy(hist_sh.at[pl.ds(sid * rows_per_sub, rows_per_sub)],
                    counts_hbm.at[cid, pl.ds(sid * rows_per_sub, rows_per_sub)])


def _sc_call(e, idx2):
    ones = jnp.ones((CHUNK, 16), jnp.float32)
    zeros = jnp.zeros((N_CODE // 16, 16), jnp.float32)
    mesh = plsc.VectorSubcoreMesh(core_axis_name="c", subcore_axis_name="s")
    k = functools.partial(
        pl.kernel,
        mesh=mesh,
        out_type=[
            jax.ShapeDtypeStruct((N_TOK, DIM), jnp.float32),
            jax.ShapeDtypeStruct((2, N_CODE, 16), jnp.float32),
        ],
        scratch_types=[
            pltpu.VMEM((NW * NCHUNK // NW, CHUNK), jnp.int32),
            pltpu.VMEM((BPW, DIM), jnp.float32),
            pltpu.VMEM((CHUNK, 16), jnp.float32),
            pltpu.VMEM_SHARED((N_CODE, 16), jnp.float32),
            pltpu.SemaphoreType.DMA,
        ],
        compiler_params=pltpu.CompilerParams(use_tc_tiling_on_sc=False),
    )(_sc_body)
    return k(e, idx2, ones, zeros)


def _final_body(x_ref, q_ref, c_ref, qst_ref, com_ref, ql_ref, pp_ref):
    xb = x_ref[...]
    qb = q_ref[...]
    dlt = qb - xb
    qst_ref[...] = xb + dlt
    sse = jnp.sum(dlt * dlt)
    el = sse / jnp.float32(N_TOK * DIM)
    com_ref[...] = jnp.reshape(COMMIT * el, (1, 1))
    ql_ref[...] = jnp.reshape(el, (1, 1))
    c = c_ref[0, :, 0:1] + c_ref[1, :, 0:1]
    p = c * jnp.float32(1.0 / N_TOK)
    ent = jnp.sum(p * jnp.log(p + 1e-8))
    pp_ref[...] = jnp.reshape(jnp.exp(-ent), (1, 1))


def _final_call(x, q, counts, interpret=False):
    s11 = jax.ShapeDtypeStruct((1, 1), jnp.float32)
    return pl.pallas_call(
        _final_body,
        out_shape=[jax.ShapeDtypeStruct((N_TOK, DIM), jnp.float32),
                   s11, s11, s11],
        interpret=interpret,
    )(x, q, counts)


def kernel(x, embedding_weight):
    idx2 = _argmin_call(x, embedding_weight)
    idx_chunks = idx2.reshape(NW * NCHUNK, CHUNK)
    q, counts = _sc_call(embedding_weight, idx_chunks)
    qst, com, ql, pp = _final_call(x, q, counts)
    return (qst, com.reshape(()), ql.reshape(()), pp.reshape(()), idx2)


# speedup vs baseline: 9.2920x; 2.6754x over previous
import functools

import jax
import jax.numpy as jnp
from jax import lax
from jax.experimental import pallas as pl
from jax.experimental.pallas import tpu as pltpu
from jax.experimental.pallas import tpu_sc as plsc

N_TOK = 9216
N_CODE = 8192
DIM = 64
COMMIT = 0.25

TM = 512
TN = 2048
NT = N_TOK // TM
NCB = N_CODE // TN

NW = 32
BPW = N_TOK // NW
CHUNK = 72
NCHUNK = BPW // CHUNK


def _argmin_body(x_ref, e_ref, idx_ref, minv_ref, mini_ref):
    cb = pl.program_id(1)

    @pl.when(cb == 0)
    def _init():
        minv_ref[...] = jnp.full((TM, 1), jnp.inf, jnp.float32)
        mini_ref[...] = jnp.zeros((TM, 1), jnp.int32)

    xb = x_ref[...]
    eb = e_ref[...]
    xn = jnp.sum(xb * xb, axis=1, keepdims=True)
    en = jnp.sum(eb * eb, axis=1)
    m = lax.dot_general(xb, eb, (((1,), (1,)), ((), ())),
                        precision=lax.Precision.DEFAULT,
                        preferred_element_type=jnp.float32)
    d = (xn + en[None, :]) - 2.0 * m

    bmin = jnp.min(d, axis=1, keepdims=True)
    ii = lax.broadcasted_iota(jnp.int32, (TM, TN), 1)
    big = jnp.int32(2**31 - 1)
    bidx = jnp.min(jnp.where(d == bmin, ii, big), axis=1, keepdims=True)
    bidx = bidx + cb * TN

    better = bmin < minv_ref[...]
    minv_ref[...] = jnp.where(better, bmin, minv_ref[...])
    mini_ref[...] = jnp.where(better, bidx, mini_ref[...])

    @pl.when(cb == NCB - 1)
    def _flush():
        idx_ref[...] = mini_ref[...]


def _argmin_call(x, e, interpret=False):
    return pl.pallas_call(
        _argmin_body,
        grid=(NT, NCB),
        in_specs=[
            pl.BlockSpec((TM, DIM), lambda t, c: (t, 0)),
            pl.BlockSpec((TN, DIM), lambda t, c: (c, 0)),
        ],
        out_specs=pl.BlockSpec((TM, 1), lambda t, c: (t, 0)),
        out_shape=jax.ShapeDtypeStruct((N_TOK, 1), jnp.int32),
        scratch_shapes=[
            pltpu.VMEM((TM, 1), jnp.float32),
            pltpu.VMEM((TM, 1), jnp.int32),
        ],
        interpret=interpret,
    )(x, e)


def _sc_body(table_hbm, idx_hbm, ones_hbm, zeros_hbm, q_hbm, counts_hbm,
             idx_v, rows_v, ones_v, hist_sh, sem):
    cid = lax.axis_index("c")
    sid = lax.axis_index("s")
    wid = sid * 2 + cid

    pltpu.sync_copy(idx_hbm.at[pl.ds(wid * NCHUNK, NCHUNK)], idx_v)
    for j in range(NCHUNK):
        pltpu.async_copy(table_hbm.at[idx_v.at[j]],
                         rows_v.at[pl.ds(j * CHUNK, CHUNK)], sem).wait()
    pltpu.sync_copy(rows_v, q_hbm.at[pl.ds(wid * BPW, BPW)])

    rows_per_sub = N_CODE // 16
    pltpu.sync_copy(zeros_hbm,
                    hist_sh.at[pl.ds(sid * rows_per_sub, rows_per_sub)])
    pltpu.sync_copy(ones_hbm, ones_v)
    plsc.subcore_barrier()
    for j in range(NCHUNK):
        pltpu.sync_copy(ones_v, hist_sh.at[idx_v.at[j]], add=True)
    plsc.subcore_barrier()
    pltpu.sync_copy(hist_sh.at[pl.ds(sid * rows_per_sub, rows_per_sub)],
                    counts_hbm.at[cid, pl.ds(sid * rows_per_sub, rows_per_sub)])


def _sc_call(e, idx2):
    ones = jnp.ones((CHUNK, 16), jnp.float32)
    zeros = jnp.zeros((N_CODE // 16, 16), jnp.float32)
    mesh = plsc.VectorSubcoreMesh(core_axis_name="c", subcore_axis_name="s")
    k = functools.partial(
        pl.kernel,
        mesh=mesh,
        out_type=[
            jax.ShapeDtypeStruct((N_TOK, DIM), jnp.float32),
            jax.ShapeDtypeStruct((2, N_CODE, 16), jnp.float32),
        ],
        scratch_types=[
            pltpu.VMEM((NW * NCHUNK // NW, CHUNK), jnp.int32),
            pltpu.VMEM((BPW, DIM), jnp.float32),
            pltpu.VMEM((CHUNK, 16), jnp.float32),
            pltpu.VMEM_SHARED((N_CODE, 16), jnp.float32),
            pltpu.SemaphoreType.DMA,
        ],
        compiler_params=pltpu.CompilerParams(use_tc_tiling_on_sc=False),
    )(_sc_body)
    return k(e, idx2, ones, zeros)


def _final_body(x_ref, q_ref, c_ref, qst_ref, com_ref, ql_ref, pp_ref):
    xb = x_ref[...]
    qb = q_ref[...]
    dlt = qb - xb
    qst_ref[...] = xb + dlt
    sse = jnp.sum(dlt * dlt)
    el = sse / jnp.float32(N_TOK * DIM)
    com_ref[...] = jnp.reshape(COMMIT * el, (1, 1))
    ql_ref[...] = jnp.reshape(el, (1, 1))
    c = c_ref[0, :, 0:1] + c_ref[1, :, 0:1]
    p = c * jnp.float32(1.0 / N_TOK)
    ent = jnp.sum(p * jnp.log(p + 1e-8))
    pp_ref[...] = jnp.reshape(jnp.exp(-ent), (1, 1))


def _final_call(x, q, counts, interpret=False):
    s11 = jax.ShapeDtypeStruct((1, 1), jnp.float32)
    return pl.pallas_call(
        _final_body,
        out_shape=[jax.ShapeDtypeStruct((N_TOK, DIM), jnp.float32),
                   s11, s11, s11],
        interpret=interpret,
    )(x, q, counts)


def kernel(x, embedding_weight):
    idx2 = _argmin_call(x, embedding_weight)
    idx_chunks = idx2.reshape(NW * NCHUNK, CHUNK)
    q, counts = _sc_call(embedding_weight, idx_chunks)
    qst, com, ql, pp = _final_call(x, q, counts)
    return (qst, com.reshape(()), ql.reshape(()), pp.reshape(()), idx2)


# speedup vs baseline: 10.2751x; 1.1058x over previous
import functools

import jax
import jax.numpy as jnp
from jax import lax
from jax.experimental import pallas as pl
from jax.experimental.pallas import tpu as pltpu
from jax.experimental.pallas import tpu_sc as plsc

N_TOK = 9216
N_CODE = 8192
DIM = 64
COMMIT = 0.25

TM = 512
TN = 8192
NT = N_TOK // TM
NCB = N_CODE // TN

NW = 32
BPW = N_TOK // NW
CHUNK = 72
NCHUNK = BPW // CHUNK


def _argmin_body(x_ref, e_ref, idx_ref, minv_ref, mini_ref):
    cb = pl.program_id(1)

    @pl.when(cb == 0)
    def _init():
        minv_ref[...] = jnp.full((TM, 1), jnp.inf, jnp.float32)
        mini_ref[...] = jnp.zeros((TM, 1), jnp.int32)

    xb = x_ref[...]
    eb = e_ref[...]
    xn = jnp.sum(xb * xb, axis=1, keepdims=True)
    en = jnp.sum(eb * eb, axis=1)
    m = lax.dot_general(xb, eb, (((1,), (1,)), ((), ())),
                        precision=lax.Precision.DEFAULT,
                        preferred_element_type=jnp.float32)
    d = (xn + en[None, :]) - 2.0 * m

    bmin = jnp.min(d, axis=1, keepdims=True)
    ii = lax.broadcasted_iota(jnp.int32, (TM, TN), 1)
    big = jnp.int32(2**31 - 1)
    bidx = jnp.min(jnp.where(d == bmin, ii, big), axis=1, keepdims=True)
    bidx = bidx + cb * TN

    better = bmin < minv_ref[...]
    minv_ref[...] = jnp.where(better, bmin, minv_ref[...])
    mini_ref[...] = jnp.where(better, bidx, mini_ref[...])

    @pl.when(cb == NCB - 1)
    def _flush():
        idx_ref[...] = mini_ref[...]


def _argmin_call(x, e, interpret=False):
    return pl.pallas_call(
        _argmin_body,
        grid=(NT, NCB),
        in_specs=[
            pl.BlockSpec((TM, DIM), lambda t, c: (t, 0)),
            pl.BlockSpec((TN, DIM), lambda t, c: (c, 0)),
        ],
        out_specs=pl.BlockSpec((TM, 1), lambda t, c: (t, 0)),
        out_shape=jax.ShapeDtypeStruct((N_TOK, 1), jnp.int32),
        scratch_shapes=[
            pltpu.VMEM((TM, 1), jnp.float32),
            pltpu.VMEM((TM, 1), jnp.int32),
        ],
        interpret=interpret,
    )(x, e)


def _sc_body(table_hbm, idx_hbm, ones_hbm, zeros_hbm, q_hbm, counts_hbm,
             idx_v, rows_v, ones_v, hist_sh, sem):
    cid = lax.axis_index("c")
    sid = lax.axis_index("s")
    wid = sid * 2 + cid

    pltpu.sync_copy(idx_hbm.at[pl.ds(wid * NCHUNK, NCHUNK)], idx_v)
    for j in range(NCHUNK):
        pltpu.async_copy(table_hbm.at[idx_v.at[j]],
                         rows_v.at[pl.ds(j * CHUNK, CHUNK)], sem).wait()
    pltpu.sync_copy(rows_v, q_hbm.at[pl.ds(wid * BPW, BPW)])

    rows_per_sub = N_CODE // 16
    pltpu.sync_copy(zeros_hbm,
                    hist_sh.at[pl.ds(sid * rows_per_sub, rows_per_sub)])
    pltpu.sync_copy(ones_hbm, ones_v)
    plsc.subcore_barrier()
    for j in range(NCHUNK):
        pltpu.sync_copy(ones_v, hist_sh.at[idx_v.at[j]], add=True)
    plsc.subcore_barrier()
    pltpu.sync_copy(hist_sh.at[pl.ds(sid * rows_per_sub, rows_per_sub)],
                    counts_hbm.at[cid, pl.ds(sid * rows_per_sub, rows_per_sub)])


def _sc_call(e, idx2):
    ones = jnp.ones((CHUNK, 16), jnp.float32)
    zeros = jnp.zeros((N_CODE // 16, 16), jnp.float32)
    mesh = plsc.VectorSubcoreMesh(core_axis_name="c", subcore_axis_name="s")
    k = functools.partial(
        pl.kernel,
        mesh=mesh,
        out_type=[
            jax.ShapeDtypeStruct((N_TOK, DIM), jnp.float32),
            jax.ShapeDtypeStruct((2, N_CODE, 16), jnp.float32),
        ],
        scratch_types=[
            pltpu.VMEM((NW * NCHUNK // NW, CHUNK), jnp.int32),
            pltpu.VMEM((BPW, DIM), jnp.float32),
            pltpu.VMEM((CHUNK, 16), jnp.float32),
            pltpu.VMEM_SHARED((N_CODE, 16), jnp.float32),
            pltpu.SemaphoreType.DMA,
        ],
        compiler_params=pltpu.CompilerParams(use_tc_tiling_on_sc=False),
    )(_sc_body)
    return k(e, idx2, ones, zeros)


def _final_body(x_ref, q_ref, c_ref, qst_ref, com_ref, ql_ref, pp_ref):
    xb = x_ref[...]
    qb = q_ref[...]
    dlt = qb - xb
    qst_ref[...] = xb + dlt
    sse = jnp.sum(dlt * dlt)
    el = sse / jnp.float32(N_TOK * DIM)
    com_ref[...] = jnp.reshape(COMMIT * el, (1, 1))
    ql_ref[...] = jnp.reshape(el, (1, 1))
    c = c_ref[0, :, 0:1] + c_ref[1, :, 0:1]
    p = c * jnp.float32(1.0 / N_TOK)
    ent = jnp.sum(p * jnp.log(p + 1e-8))
    pp_ref[...] = jnp.reshape(jnp.exp(-ent), (1, 1))


def _final_call(x, q, counts, interpret=False):
    s11 = jax.ShapeDtypeStruct((1, 1), jnp.float32)
    return pl.pallas_call(
        _final_body,
        out_shape=[jax.ShapeDtypeStruct((N_TOK, DIM), jnp.float32),
                   s11, s11, s11],
        interpret=interpret,
    )(x, q, counts)


def kernel(x, embedding_weight):
    idx2 = _argmin_call(x, embedding_weight)
    idx_chunks = idx2.reshape(NW * NCHUNK, CHUNK)
    q, counts = _sc_call(embedding_weight, idx_chunks)
    qst, com, ql, pp = _final_call(x, q, counts)
    return (qst, com.reshape(()), ql.reshape(()), pp.reshape(()), idx2)


# speedup vs baseline: 10.6338x; 1.0349x over previous
import functools

import jax
import jax.numpy as jnp
from jax import lax
from jax.experimental import pallas as pl
from jax.experimental.pallas import tpu as pltpu
from jax.experimental.pallas import tpu_sc as plsc

N_TOK = 9216
N_CODE = 8192
DIM = 64
COMMIT = 0.25

TM = 1024
TN = 8192
NT = N_TOK // TM
NCB = N_CODE // TN

NW = 32
BPW = N_TOK // NW
CHUNK = 72
NCHUNK = BPW // CHUNK


def _argmin_body(x_ref, e_ref, idx_ref, minv_ref, mini_ref):
    cb = pl.program_id(1)

    @pl.when(cb == 0)
    def _init():
        minv_ref[...] = jnp.full((TM, 1), jnp.inf, jnp.float32)
        mini_ref[...] = jnp.zeros((TM, 1), jnp.int32)

    xb = x_ref[...]
    eb = e_ref[...]
    xn = jnp.sum(xb * xb, axis=1, keepdims=True)
    en = jnp.sum(eb * eb, axis=1)
    m = lax.dot_general(xb, eb, (((1,), (1,)), ((), ())),
                        precision=lax.Precision.DEFAULT,
                        preferred_element_type=jnp.float32)
    d = (xn + en[None, :]) - 2.0 * m

    bmin = jnp.min(d, axis=1, keepdims=True)
    ii = lax.broadcasted_iota(jnp.int32, (TM, TN), 1)
    big = jnp.int32(2**31 - 1)
    bidx = jnp.min(jnp.where(d == bmin, ii, big), axis=1, keepdims=True)
    bidx = bidx + cb * TN

    better = bmin < minv_ref[...]
    minv_ref[...] = jnp.where(better, bmin, minv_ref[...])
    mini_ref[...] = jnp.where(better, bidx, mini_ref[...])

    @pl.when(cb == NCB - 1)
    def _flush():
        idx_ref[...] = mini_ref[...]


def _argmin_call(x, e, interpret=False):
    return pl.pallas_call(
        _argmin_body,
        grid=(NT, NCB),
        in_specs=[
            pl.BlockSpec((TM, DIM), lambda t, c: (t, 0)),
            pl.BlockSpec((TN, DIM), lambda t, c: (c, 0)),
        ],
        out_specs=pl.BlockSpec((TM, 1), lambda t, c: (t, 0)),
        out_shape=jax.ShapeDtypeStruct((N_TOK, 1), jnp.int32),
        scratch_shapes=[
            pltpu.VMEM((TM, 1), jnp.float32),
            pltpu.VMEM((TM, 1), jnp.int32),
        ],
        interpret=interpret,
    )(x, e)


def _sc_body(table_hbm, idx_hbm, ones_hbm, zeros_hbm, q_hbm, counts_hbm,
             idx_v, rows_v, ones_v, hist_sh, sem):
    cid = lax.axis_index("c")
    sid = lax.axis_index("s")
    wid = sid * 2 + cid

    pltpu.sync_copy(idx_hbm.at[pl.ds(wid * NCHUNK, NCHUNK)], idx_v)
    for j in range(NCHUNK):
        pltpu.async_copy(table_hbm.at[idx_v.at[j]],
                         rows_v.at[pl.ds(j * CHUNK, CHUNK)], sem).wait()
    pltpu.sync_copy(rows_v, q_hbm.at[pl.ds(wid * BPW, BPW)])

    rows_per_sub = N_CODE // 16
    pltpu.sync_copy(zeros_hbm,
                    hist_sh.at[pl.ds(sid * rows_per_sub, rows_per_sub)])
    pltpu.sync_copy(ones_hbm, ones_v)
    plsc.subcore_barrier()
    for j in range(NCHUNK):
        pltpu.sync_copy(ones_v, hist_sh.at[idx_v.at[j]], add=True)
    plsc.subcore_barrier()
    pltpu.sync_copy(hist_sh.at[pl.ds(sid * rows_per_sub, rows_per_sub)],
                    counts_hbm.at[cid, pl.ds(sid * rows_per_sub, rows_per_sub)])


def _sc_call(e, idx2):
    ones = jnp.ones((CHUNK, 16), jnp.float32)
    zeros = jnp.zeros((N_CODE // 16, 16), jnp.float32)
    mesh = plsc.VectorSubcoreMesh(core_axis_name="c", subcore_axis_name="s")
    k = functools.partial(
        pl.kernel,
        mesh=mesh,
        out_type=[
            jax.ShapeDtypeStruct((N_TOK, DIM), jnp.float32),
            jax.ShapeDtypeStruct((2, N_CODE, 16), jnp.float32),
        ],
        scratch_types=[
            pltpu.VMEM((NW * NCHUNK // NW, CHUNK), jnp.int32),
            pltpu.VMEM((BPW, DIM), jnp.float32),
            pltpu.VMEM((CHUNK, 16), jnp.float32),
            pltpu.VMEM_SHARED((N_CODE, 16), jnp.float32),
            pltpu.SemaphoreType.DMA,
        ],
        compiler_params=pltpu.CompilerParams(use_tc_tiling_on_sc=False),
    )(_sc_body)
    return k(e, idx2, ones, zeros)


def _final_body(x_ref, q_ref, c_ref, qst_ref, com_ref, ql_ref, pp_ref):
    xb = x_ref[...]
    qb = q_ref[...]
    dlt = qb - xb
    qst_ref[...] = xb + dlt
    sse = jnp.sum(dlt * dlt)
    el = sse / jnp.float32(N_TOK * DIM)
    com_ref[...] = jnp.reshape(COMMIT * el, (1, 1))
    ql_ref[...] = jnp.reshape(el, (1, 1))
    c = c_ref[0, :, 0:1] + c_ref[1, :, 0:1]
    p = c * jnp.float32(1.0 / N_TOK)
    ent = jnp.sum(p * jnp.log(p + 1e-8))
    pp_ref[...] = jnp.reshape(jnp.exp(-ent), (1, 1))


def _final_call(x, q, counts, interpret=False):
    s11 = jax.ShapeDtypeStruct((1, 1), jnp.float32)
    return pl.pallas_call(
        _final_body,
        out_shape=[jax.ShapeDtypeStruct((N_TOK, DIM), jnp.float32),
                   s11, s11, s11],
        interpret=interpret,
    )(x, q, counts)


def kernel(x, embedding_weight):
    idx2 = _argmin_call(x, embedding_weight)
    idx_chunks = idx2.reshape(NW * NCHUNK, CHUNK)
    q, counts = _sc_call(embedding_weight, idx_chunks)
    qst, com, ql, pp = _final_call(x, q, counts)
    return (qst, com.reshape(()), ql.reshape(()), pp.reshape(()), idx2)
